# baseline (speedup 1.0000x reference)
import functools
import jax
import jax.numpy as jnp
from jax import lax
from jax.experimental import pallas as pl
from jax.experimental.pallas import tpu as pltpu
from jax.experimental.pallas import tpu_sc as plsc

L = 16
CHUNK = 128


def _cf_kernel_body(nw, bpw, kd, uid_hbm, iid_hbm, ut_hbm, it_hbm, w_hbm,
                    b_hbm, out_hbm, uidx_v, iidx_v, urows_v, irows_v,
                    w_v, b_v, tmp_v, out_v, sem):
    c = lax.axis_index("c")
    s = lax.axis_index("s")
    wid = s * 2 + c
    base = wid * bpw

    pltpu.sync_copy(uid_hbm.at[wid], uidx_v)
    pltpu.sync_copy(iid_hbm.at[wid], iidx_v)
    pltpu.sync_copy(w_hbm, w_v)
    pltpu.sync_copy(b_hbm, b_v)

    nchunks = bpw // CHUNK
    for ci in range(nchunks):
        pltpu.async_copy(ut_hbm.at[uidx_v.at[ci]],
                         urows_v.at[pl.ds(ci * CHUNK, CHUNK)], sem).wait()
        pltpu.async_copy(it_hbm.at[iidx_v.at[ci]],
                         irows_v.at[pl.ds(ci * CHUNK, CHUNK)], sem).wait()

    wk = [w_v[pl.ds(k * L, L)] for k in range(kd)]
    bvec = b_v[...]
    iota = lax.iota(jnp.int32, L)

    def group_body(g, _):
        out_vec = bvec
        for e in range(L):
            i = g * L + e
            acc = (urows_v[i, pl.ds(0, L)] * irows_v[i, pl.ds(0, L)]) * wk[0]
            for k in range(1, kd):
                acc = acc + (urows_v[i, pl.ds(k * L, L)]
                             * irows_v[i, pl.ds(k * L, L)]) * wk[k]
            for sh in (8, 4, 2, 1):
                tmp_v[pl.ds(0, L)] = acc
                tmp_v[pl.ds(L, L)] = acc
                acc = acc + tmp_v[pl.ds(sh, L)]
            out_vec = jnp.where(iota == e, acc, out_vec)
        out_v[pl.ds(g * L, L)] = out_vec
        return 0

    lax.fori_loop(0, bpw // L, group_body, 0)
    pltpu.sync_copy(out_v, out_hbm.at[pl.ds(base, bpw)])


def kernel(user_ids, item_ids, user_table, item_table, fc_w, fc_b):
    B = user_ids.shape[0]
    H = user_table.shape[1]
    kd = H // L
    nw = 32
    bpw = B // nw

    uid = user_ids.astype(jnp.int32).reshape(nw, bpw // CHUNK, CHUNK)
    iid = item_ids.astype(jnp.int32).reshape(nw, bpw // CHUNK, CHUNK)
    ut = user_table
    it = item_table
    w = fc_w.reshape(H)
    b = jnp.broadcast_to(fc_b.reshape(1), (L,))

    mesh = plsc.VectorSubcoreMesh(core_axis_name="c", subcore_axis_name="s")
    out = pl.kernel(
        functools.partial(_cf_kernel_body, nw, bpw, kd),
        mesh=mesh,
        compiler_params=pltpu.CompilerParams(use_tc_tiling_on_sc=False),
        out_type=jax.ShapeDtypeStruct((B,), jnp.float32),
        scratch_types=[
            pltpu.VMEM((bpw // CHUNK, CHUNK), jnp.int32),
            pltpu.VMEM((bpw // CHUNK, CHUNK), jnp.int32),
            pltpu.VMEM((bpw, kd * L), jnp.float32),
            pltpu.VMEM((bpw, kd * L), jnp.float32),
            pltpu.VMEM((kd * L,), jnp.float32),
            pltpu.VMEM((L,), jnp.float32),
            pltpu.VMEM((2 * L,), jnp.float32),
            pltpu.VMEM((bpw,), jnp.float32),
            pltpu.SemaphoreType.DMA,
        ],
    )(uid, iid, ut, it, w, b)
    return out.reshape(B, 1)


# speedup vs baseline: 1.0204x; 1.0204x over previous
import functools
import jax
import jax.numpy as jnp
from jax import lax
from jax.experimental import pallas as pl
from jax.experimental.pallas import tpu as pltpu
from jax.experimental.pallas import tpu_sc as plsc

L = 16
CHUNK = 128
UNROLL = 16


def _cf_kernel_body(nw, bpw, kd, uid_hbm, iid_hbm, ut_hbm, it_hbm, w_hbm,
                    b_hbm, out_hbm, uidx_v, iidx_v, urows_v, irows_v,
                    w_v, b_v, out_v, sem):
    c = lax.axis_index("c")
    s = lax.axis_index("s")
    wid = s * 2 + c
    base = wid * bpw

    pltpu.sync_copy(uid_hbm.at[wid], uidx_v)
    pltpu.sync_copy(iid_hbm.at[wid], iidx_v)
    pltpu.sync_copy(w_hbm, w_v)
    pltpu.sync_copy(b_hbm, b_v)

    nchunks = bpw // CHUNK
    handles = []
    for ci in range(nchunks):
        handles.append(pltpu.async_copy(
            ut_hbm.at[uidx_v.at[ci]],
            urows_v.at[pl.ds(ci * CHUNK, CHUNK)], sem))
        handles.append(pltpu.async_copy(
            it_hbm.at[iidx_v.at[ci]],
            irows_v.at[pl.ds(ci * CHUNK, CHUNK)], sem))
    for h in handles:
        h.wait()

    wk = [w_v[pl.ds(k * L, L)] for k in range(kd)]
    iota = lax.iota(jnp.int32, L)
    bias = b_v[...] * (1.0 / L)
    dn = lax.GatherDimensionNumbers(offset_dims=(),
                                    collapsed_slice_dims=(0,),
                                    start_index_map=(0,))
    perms = [(iota ^ sh).reshape(L, 1) for sh in (8, 4, 2, 1)]

    def hsum(v):
        for perm in perms:
            v = v + lax.gather(v, perm, dn, (1,),
                               mode=lax.GatherScatterMode.PROMISE_IN_BOUNDS)
        return v

    def group_body(g, _):
        out_vec = jnp.zeros((L,), jnp.float32)
        for e in range(UNROLL):
            i = g * UNROLL + e
            acc = bias + (urows_v[i, pl.ds(0, L)]
                          * irows_v[i, pl.ds(0, L)]) * wk[0]
            for k in range(1, kd):
                acc = acc + (urows_v[i, pl.ds(k * L, L)]
                             * irows_v[i, pl.ds(k * L, L)]) * wk[k]
            out_vec = jnp.where(iota == e, hsum(acc), out_vec)
        out_v[pl.ds(g * UNROLL, UNROLL)] = out_vec
        return 0

    lax.fori_loop(0, bpw // UNROLL, group_body, 0)
    pltpu.sync_copy(out_v, out_hbm.at[pl.ds(base, bpw)])


def kernel(user_ids, item_ids, user_table, item_table, fc_w, fc_b):
    B = user_ids.shape[0]
    H = user_table.shape[1]
    kd = H // L
    nw = 32
    bpw = B // nw

    uid = user_ids.astype(jnp.int32).reshape(nw, bpw // CHUNK, CHUNK)
    iid = item_ids.astype(jnp.int32).reshape(nw, bpw // CHUNK, CHUNK)
    ut = user_table
    it = item_table
    w = fc_w.reshape(H)
    b = jnp.broadcast_to(fc_b.reshape(1), (L,))

    mesh = plsc.VectorSubcoreMesh(core_axis_name="c", subcore_axis_name="s")
    out = pl.kernel(
        functools.partial(_cf_kernel_body, nw, bpw, kd),
        mesh=mesh,
        compiler_params=pltpu.CompilerParams(use_tc_tiling_on_sc=False),
        out_type=jax.ShapeDtypeStruct((B,), jnp.float32),
        scratch_types=[
            pltpu.VMEM((bpw // CHUNK, CHUNK), jnp.int32),
            pltpu.VMEM((bpw // CHUNK, CHUNK), jnp.int32),
            pltpu.VMEM((bpw, kd * L), jnp.float32),
            pltpu.VMEM((bpw, kd * L), jnp.float32),
            pltpu.VMEM((kd * L,), jnp.float32),
            pltpu.VMEM((L,), jnp.float32),
            pltpu.VMEM((bpw,), jnp.float32),
            pltpu.SemaphoreType.DMA,
        ],
    )(uid, iid, ut, it, w, b)
    return out.reshape(B, 1)


# speedup vs baseline: 1.0214x; 1.0009x over previous
import functools
import jax
import jax.numpy as jnp
from jax import lax
from jax.experimental import pallas as pl
from jax.experimental.pallas import tpu as pltpu
from jax.experimental.pallas import tpu_sc as plsc

L = 16
CH = 128


def _cf_kernel_body(bpw, kd, uid_hbm, iid_hbm, ut_hbm, it_hbm, w_hbm,
                    b_hbm, out_hbm, uidx_v, iidx_v, urows_v, irows_v,
                    w_v, b_v, out_v, sem):
    c = lax.axis_index("c")
    s = lax.axis_index("s")
    wid = s * 2 + c

    pltpu.sync_copy(uid_hbm.at[wid], uidx_v)
    pltpu.sync_copy(iid_hbm.at[wid], iidx_v)
    pltpu.sync_copy(w_hbm, w_v)
    pltpu.sync_copy(b_hbm, b_v)

    handles = []
    for j in range(bpw // CH):
        handles.append(pltpu.async_copy(
            ut_hbm.at[uidx_v.at[j]], urows_v.at[pl.ds(j * CH, CH)], sem))
        handles.append(pltpu.async_copy(
            it_hbm.at[iidx_v.at[j]], irows_v.at[pl.ds(j * CH, CH)], sem))
    for h in handles:
        h.wait()

    wk = [w_v[pl.ds(k * L, L)] for k in range(kd)]
    iota = lax.iota(jnp.int32, L)
    bias = b_v[...] * (1.0 / L)
    dn = lax.GatherDimensionNumbers(offset_dims=(),
                                    collapsed_slice_dims=(0,),
                                    start_index_map=(0,))
    perms = [(iota ^ (1 << t)).reshape(L, 1) for t in range(4)]
    masks = [(iota & (1 << t)) != 0 for t in range(4)]

    def shuf(v, t):
        return lax.gather(v, perms[t], dn, (1,),
                          mode=lax.GatherScatterMode.PROMISE_IN_BOUNDS)

    def group_body(g, _):
        base = g * L
        accs = []
        for j in range(L):
            e = base + j
            acc = bias + (urows_v[e, pl.ds(0, L)]
                          * irows_v[e, pl.ds(0, L)]) * wk[0]
            for k in range(1, kd):
                acc = acc + (urows_v[e, pl.ds(k * L, L)]
                             * irows_v[e, pl.ds(k * L, L)]) * wk[k]
            accs.append(acc)
        vs = accs
        for t in range(4):
            nxt = []
            for p in range(len(vs) // 2):
                a, b = vs[2 * p], vs[2 * p + 1]
                nxt.append(jnp.where(masks[t], b + shuf(b, t), a + shuf(a, t)))
            vs = nxt
        out_v[pl.ds(base, L)] = vs[0]
        return 0

    lax.fori_loop(0, bpw // L, group_body, 0)
    pltpu.sync_copy(out_v, out_hbm.at[pl.ds(wid * bpw, bpw)])


def kernel(user_ids, item_ids, user_table, item_table, fc_w, fc_b):
    B = user_ids.shape[0]
    H = user_table.shape[1]
    kd = H // L
    nw = 32
    bpw = B // nw

    uid = user_ids.astype(jnp.int32).reshape(nw, bpw // CH, CH)
    iid = item_ids.astype(jnp.int32).reshape(nw, bpw // CH, CH)
    w = fc_w.reshape(H)
    b = jnp.broadcast_to(fc_b.reshape(1), (L,))

    mesh = plsc.VectorSubcoreMesh(core_axis_name="c", subcore_axis_name="s")
    out = pl.kernel(
        functools.partial(_cf_kernel_body, bpw, kd),
        mesh=mesh,
        compiler_params=pltpu.CompilerParams(use_tc_tiling_on_sc=False),
        out_type=jax.ShapeDtypeStruct((B,), jnp.float32),
        scratch_types=[
            pltpu.VMEM((bpw // CH, CH), jnp.int32),
            pltpu.VMEM((bpw // CH, CH), jnp.int32),
            pltpu.VMEM((bpw, H), jnp.float32),
            pltpu.VMEM((bpw, H), jnp.float32),
            pltpu.VMEM((kd * L,), jnp.float32),
            pltpu.VMEM((L,), jnp.float32),
            pltpu.VMEM((bpw,), jnp.float32),
            pltpu.SemaphoreType.DMA,
        ],
    )(uid, iid, user_table, item_table, w, b)
    return out.reshape(B, 1)
